# baseline (speedup 1.0000x reference)
import functools

import jax
import jax.numpy as jnp
from jax import lax
from jax.experimental import pallas as pl
from jax.experimental.pallas import tpu as pltpu
from jax.experimental.pallas import tpu_sc as plsc

WE_DIM = 64
EE_DIM = 64
OUT_DIM = WE_DIM + EE_DIM
TAB_DIM = 128

NC = 2
NS = 16
NW = NC * NS

SUB = 128
CHUNK = 128
NSUB = CHUNK // SUB

L16 = 16


TBLK = 512


def _make_transpose(v: int, d: int):
    v_pad = ((v + TBLK - 1) // TBLK) * TBLK

    @functools.partial(
        pl.pallas_call,
        grid=(v_pad // TBLK,),
        in_specs=[pl.BlockSpec((d, TBLK), lambda i: (0, i))],
        out_specs=pl.BlockSpec((TBLK, TAB_DIM), lambda i: (i, 0)),
        out_shape=jax.ShapeDtypeStruct((v_pad, TAB_DIM), jnp.float32),
    )
    def transpose_kernel(in_ref, out_ref):
        out_ref[:, 0:d] = jnp.transpose(in_ref[...], (1, 0))

    return transpose_kernel


def _make_gather(n_rows: int, wv_pad: int, ev_pad: int):
    rows_per_w = n_rows // NW
    n_chunks = rows_per_w // CHUNK
    n_pairs = n_chunks // 2
    assert n_chunks % 2 == 0 and n_pairs >= 2
    mesh = plsc.VectorSubcoreMesh(core_axis_name="c", subcore_axis_name="s")

    @functools.partial(
        pl.kernel,
        mesh=mesh,
        compiler_params=pltpu.CompilerParams(use_tc_tiling_on_sc=False),
        out_type=jax.ShapeDtypeStruct((n_rows, OUT_DIM), jnp.float32),
        scratch_types=[
            pltpu.VMEM((2, NSUB, SUB), jnp.int32),
            pltpu.VMEM((2, NSUB, SUB), jnp.int32),
            pltpu.VMEM((2, CHUNK, TAB_DIM), jnp.float32),
            pltpu.VMEM((2, CHUNK, TAB_DIM), jnp.float32),
            pltpu.SemaphoreType.DMA,
            pltpu.SemaphoreType.DMA,
            pltpu.SemaphoreType.DMA,
            pltpu.SemaphoreType.DMA,
        ],
    )
    def gather_kernel(widx_hbm, eidx_hbm, wtab_hbm, etab_hbm, out_hbm,
                      widx_v, eidx_v, wrows_v, erows_v,
                      gsem0, gsem1, ssem0, ssem1):
        wid = lax.axis_index("s") * NC + lax.axis_index("c")
        base = wid * rows_per_w
        idx_row0 = wid * (rows_per_w // SUB)
        gsem = (gsem0, gsem1)
        ssem = (ssem0, ssem1)

        def load_idx(c, b):
            crow = idx_row0 + c * NSUB
            pltpu.sync_copy(widx_hbm.at[pl.ds(crow, NSUB)], widx_v.at[b])
            pltpu.sync_copy(eidx_hbm.at[pl.ds(crow, NSUB)], eidx_v.at[b])

        def fire(c, b):
            for j in range(NSUB):
                pltpu.async_copy(wtab_hbm.at[widx_v.at[b, j]],
                                 wrows_v.at[b, pl.ds(j * SUB, SUB)], gsem[b])
                pltpu.async_copy(etab_hbm.at[eidx_v.at[b, j]],
                                 erows_v.at[b, pl.ds(j * SUB, SUB)], gsem[b])

        def wait_g(b):
            pltpu.make_async_copy(wtab_hbm.at[pl.ds(0, CHUNK)],
                                  wrows_v.at[b], gsem[b]).wait()
            pltpu.make_async_copy(wtab_hbm.at[pl.ds(0, CHUNK)],
                                  erows_v.at[b], gsem[b]).wait()

        def store(c, b):
            cbase = base + c * CHUNK
            pltpu.async_copy(wrows_v.at[b, :, pl.ds(0, WE_DIM)],
                             out_hbm.at[pl.ds(cbase, CHUNK), pl.ds(0, WE_DIM)],
                             ssem[b])
            pltpu.async_copy(erows_v.at[b, :, pl.ds(0, EE_DIM)],
                             out_hbm.at[pl.ds(cbase, CHUNK), pl.ds(WE_DIM, EE_DIM)],
                             ssem[b])

        def wait_s(b):
            pltpu.make_async_copy(wtab_hbm.at[pl.ds(0, CHUNK // 2)],
                                  wrows_v.at[b, pl.ds(0, CHUNK // 2)],
                                  ssem[b]).wait()
            pltpu.make_async_copy(wtab_hbm.at[pl.ds(0, CHUNK // 2)],
                                  erows_v.at[b, pl.ds(0, CHUNK // 2)],
                                  ssem[b]).wait()

        load_idx(0, 0)
        fire(0, 0)
        load_idx(1, 1)
        fire(1, 1)
        wait_g(0)
        store(0, 0)
        wait_s(0)
        load_idx(2, 0)
        fire(2, 0)
        wait_g(1)
        store(1, 1)

        def body(p, carry):
            c0 = 2 * p
            c1 = c0 + 1
            wait_s(1)
            load_idx(c1, 1)
            fire(c1, 1)
            wait_g(0)
            store(c0, 0)
            wait_s(0)
            load_idx(c0 + 2, 0)
            fire(c0 + 2, 0)
            wait_g(1)
            store(c1, 1)
            return carry

        lax.fori_loop(1, n_pairs - 1, body, 0, unroll=False)

        c0 = n_chunks - 2
        wait_s(1)
        load_idx(c0 + 1, 1)
        fire(c0 + 1, 1)
        wait_g(0)
        store(c0, 0)
        wait_g(1)
        store(c0 + 1, 1)
        wait_s(0)
        wait_s(1)

    return gather_kernel


def kernel(lctx_words, rctx_words, lctx_entities, rctx_entities,
           word_table, entity_table):
    b, l = lctx_words.shape
    n_rows = 2 * b * l
    wv, wd = word_table.shape
    ev, ed = entity_table.shape

    widx = jnp.concatenate(
        [rctx_words.reshape(-1), lctx_words.reshape(-1)]
    ).astype(jnp.int32).reshape(n_rows // SUB, SUB)
    eidx = jnp.concatenate(
        [rctx_entities.reshape(-1), lctx_entities.reshape(-1)]
    ).astype(jnp.int32).reshape(n_rows // SUB, SUB)

    wtab = _make_transpose(wv, wd)(word_table.T)
    etab = _make_transpose(ev, ed)(entity_table.T)
    out = _make_gather(n_rows, wtab.shape[0], etab.shape[0])(
        widx, eidx, wtab, etab)
    return out.reshape(2, b, l, OUT_DIM)


# speedup vs baseline: 1.2863x; 1.2863x over previous
import functools

import jax
import jax.numpy as jnp
from jax import lax
from jax.experimental import pallas as pl
from jax.experimental.pallas import tpu as pltpu
from jax.experimental.pallas import tpu_sc as plsc

WE_DIM = 64
EE_DIM = 64
OUT_DIM = WE_DIM + EE_DIM
TAB_DIM = 128

NC = 2
NS = 16
NW = NC * NS

SUB = 128
CHUNK = 128
NSUB = CHUNK // SUB

L16 = 16


TBLK = 1024


def _make_transpose(v: int, d: int):
    v_pad = ((v + TBLK - 1) // TBLK) * TBLK

    @functools.partial(
        pl.pallas_call,
        grid=(v_pad // TBLK,),
        in_specs=[pl.BlockSpec((d, TBLK), lambda i: (0, i))],
        out_specs=pl.BlockSpec((TBLK, TAB_DIM), lambda i: (i, 0)),
        out_shape=jax.ShapeDtypeStruct((v_pad, TAB_DIM), jnp.float32),
    )
    def transpose_kernel(in_ref, out_ref):
        rows = lax.broadcasted_iota(jnp.int32, (d, d), 0)
        cols = lax.broadcasted_iota(jnp.int32, (d, d), 1)
        eye = jnp.where(rows == cols, 1.0, 0.0).astype(jnp.float32)
        out_ref[:, 0:d] = lax.dot_general(
            in_ref[...], eye, (((0,), (0,)), ((), ())),
            precision=lax.Precision.HIGHEST,
            preferred_element_type=jnp.float32)

    return transpose_kernel


def _make_gather(n_rows: int, wv_pad: int, ev_pad: int):
    rows_per_w = n_rows // NW
    n_chunks = rows_per_w // CHUNK
    n_pairs = n_chunks // 2
    assert n_chunks % 2 == 0 and n_pairs >= 2
    mesh = plsc.VectorSubcoreMesh(core_axis_name="c", subcore_axis_name="s")

    @functools.partial(
        pl.kernel,
        mesh=mesh,
        compiler_params=pltpu.CompilerParams(use_tc_tiling_on_sc=False),
        out_type=jax.ShapeDtypeStruct((n_rows, OUT_DIM), jnp.float32),
        scratch_types=[
            pltpu.VMEM((2, NSUB, SUB), jnp.int32),
            pltpu.VMEM((2, NSUB, SUB), jnp.int32),
            pltpu.VMEM((2, CHUNK, TAB_DIM), jnp.float32),
            pltpu.VMEM((2, CHUNK, TAB_DIM), jnp.float32),
            pltpu.SemaphoreType.DMA,
            pltpu.SemaphoreType.DMA,
            pltpu.SemaphoreType.DMA,
            pltpu.SemaphoreType.DMA,
        ],
    )
    def gather_kernel(widx_hbm, eidx_hbm, wtab_hbm, etab_hbm, out_hbm,
                      widx_v, eidx_v, wrows_v, erows_v,
                      gsem0, gsem1, ssem0, ssem1):
        wid = lax.axis_index("s") * NC + lax.axis_index("c")
        base = wid * rows_per_w
        idx_row0 = wid * (rows_per_w // SUB)
        gsem = (gsem0, gsem1)
        ssem = (ssem0, ssem1)

        def load_idx(c, b):
            crow = idx_row0 + c * NSUB
            pltpu.sync_copy(widx_hbm.at[pl.ds(crow, NSUB)], widx_v.at[b])
            pltpu.sync_copy(eidx_hbm.at[pl.ds(crow, NSUB)], eidx_v.at[b])

        def fire(c, b):
            for j in range(NSUB):
                pltpu.async_copy(wtab_hbm.at[widx_v.at[b, j]],
                                 wrows_v.at[b, pl.ds(j * SUB, SUB)], gsem[b])
                pltpu.async_copy(etab_hbm.at[eidx_v.at[b, j]],
                                 erows_v.at[b, pl.ds(j * SUB, SUB)], gsem[b])

        def wait_g(b):
            pltpu.make_async_copy(wtab_hbm.at[pl.ds(0, CHUNK)],
                                  wrows_v.at[b], gsem[b]).wait()
            pltpu.make_async_copy(wtab_hbm.at[pl.ds(0, CHUNK)],
                                  erows_v.at[b], gsem[b]).wait()

        def store(c, b):
            cbase = base + c * CHUNK
            pltpu.async_copy(wrows_v.at[b, :, pl.ds(0, WE_DIM)],
                             out_hbm.at[pl.ds(cbase, CHUNK), pl.ds(0, WE_DIM)],
                             ssem[b])
            pltpu.async_copy(erows_v.at[b, :, pl.ds(0, EE_DIM)],
                             out_hbm.at[pl.ds(cbase, CHUNK), pl.ds(WE_DIM, EE_DIM)],
                             ssem[b])

        def wait_s(b):
            pltpu.make_async_copy(wtab_hbm.at[pl.ds(0, CHUNK // 2)],
                                  wrows_v.at[b, pl.ds(0, CHUNK // 2)],
                                  ssem[b]).wait()
            pltpu.make_async_copy(wtab_hbm.at[pl.ds(0, CHUNK // 2)],
                                  erows_v.at[b, pl.ds(0, CHUNK // 2)],
                                  ssem[b]).wait()

        load_idx(0, 0)
        fire(0, 0)
        load_idx(1, 1)
        fire(1, 1)
        wait_g(0)
        store(0, 0)
        wait_s(0)
        load_idx(2, 0)
        fire(2, 0)
        wait_g(1)
        store(1, 1)

        def body(p, carry):
            c0 = 2 * p
            c1 = c0 + 1
            wait_s(1)
            load_idx(c1, 1)
            fire(c1, 1)
            wait_g(0)
            store(c0, 0)
            wait_s(0)
            load_idx(c0 + 2, 0)
            fire(c0 + 2, 0)
            wait_g(1)
            store(c1, 1)
            return carry

        lax.fori_loop(1, n_pairs - 1, body, 0, unroll=False)

        c0 = n_chunks - 2
        wait_s(1)
        load_idx(c0 + 1, 1)
        fire(c0 + 1, 1)
        wait_g(0)
        store(c0, 0)
        wait_g(1)
        store(c0 + 1, 1)
        wait_s(0)
        wait_s(1)

    return gather_kernel


def kernel(lctx_words, rctx_words, lctx_entities, rctx_entities,
           word_table, entity_table):
    b, l = lctx_words.shape
    n_rows = 2 * b * l
    wv, wd = word_table.shape
    ev, ed = entity_table.shape

    widx = jnp.concatenate(
        [rctx_words.reshape(-1), lctx_words.reshape(-1)]
    ).astype(jnp.int32).reshape(n_rows // SUB, SUB)
    eidx = jnp.concatenate(
        [rctx_entities.reshape(-1), lctx_entities.reshape(-1)]
    ).astype(jnp.int32).reshape(n_rows // SUB, SUB)

    wtab = _make_transpose(wv, wd)(word_table.T)
    etab = _make_transpose(ev, ed)(entity_table.T)
    out = _make_gather(n_rows, wtab.shape[0], etab.shape[0])(
        widx, eidx, wtab, etab)
    return out.reshape(2, b, l, OUT_DIM)


# speedup vs baseline: 1.9359x; 1.5050x over previous
import functools

import jax
import jax.numpy as jnp
from jax import lax
from jax.experimental import pallas as pl
from jax.experimental.pallas import tpu as pltpu
from jax.experimental.pallas import tpu_sc as plsc

WE_DIM = 64
EE_DIM = 64
OUT_DIM = WE_DIM + EE_DIM
TAB_DIM = 128

NC = 2
NS = 16
NW = NC * NS

SUB = 128
CHUNK = 128
NSUB = CHUNK // SUB

L16 = 16


TBLK = 4096


def _make_transpose(v: int, d: int):
    v_pad = ((v + TBLK - 1) // TBLK) * TBLK

    @functools.partial(
        pl.pallas_call,
        grid=(v_pad // TBLK,),
        in_specs=[pl.BlockSpec((d, TBLK), lambda i: (0, i))],
        out_specs=pl.BlockSpec((TBLK, TAB_DIM), lambda i: (i, 0)),
        out_shape=jax.ShapeDtypeStruct((v_pad, TAB_DIM), jnp.float32),
    )
    def transpose_kernel(in_ref, out_ref):
        rows = lax.broadcasted_iota(jnp.int32, (d, d), 0)
        cols = lax.broadcasted_iota(jnp.int32, (d, d), 1)
        eye = jnp.where(rows == cols, 1.0, 0.0).astype(jnp.float32)
        out_ref[:, 0:d] = lax.dot_general(
            in_ref[...], eye, (((0,), (0,)), ((), ())),
            precision=lax.Precision.HIGHEST,
            preferred_element_type=jnp.float32)

    return transpose_kernel


def _make_gather(n_rows: int, wv_pad: int, ev_pad: int):
    rows_per_w = n_rows // NW
    n_chunks = rows_per_w // CHUNK
    n_pairs = n_chunks // 2
    assert n_chunks % 2 == 0 and n_pairs >= 2
    mesh = plsc.VectorSubcoreMesh(core_axis_name="c", subcore_axis_name="s")

    @functools.partial(
        pl.kernel,
        mesh=mesh,
        compiler_params=pltpu.CompilerParams(use_tc_tiling_on_sc=False),
        out_type=jax.ShapeDtypeStruct((n_rows, OUT_DIM), jnp.float32),
        scratch_types=[
            pltpu.VMEM((2, NSUB, SUB), jnp.int32),
            pltpu.VMEM((2, NSUB, SUB), jnp.int32),
            pltpu.VMEM((2, CHUNK, TAB_DIM), jnp.float32),
            pltpu.VMEM((2, CHUNK, TAB_DIM), jnp.float32),
            pltpu.SemaphoreType.DMA,
            pltpu.SemaphoreType.DMA,
            pltpu.SemaphoreType.DMA,
            pltpu.SemaphoreType.DMA,
        ],
    )
    def gather_kernel(widx_hbm, eidx_hbm, wtab_hbm, etab_hbm, out_hbm,
                      widx_v, eidx_v, wrows_v, erows_v,
                      gsem0, gsem1, ssem0, ssem1):
        wid = lax.axis_index("s") * NC + lax.axis_index("c")
        base = wid * rows_per_w
        idx_row0 = wid * (rows_per_w // SUB)
        gsem = (gsem0, gsem1)
        ssem = (ssem0, ssem1)

        def load_idx(c, b):
            crow = idx_row0 + c * NSUB
            pltpu.sync_copy(widx_hbm.at[pl.ds(crow, NSUB)], widx_v.at[b])
            pltpu.sync_copy(eidx_hbm.at[pl.ds(crow, NSUB)], eidx_v.at[b])

        def fire(c, b):
            for j in range(NSUB):
                pltpu.async_copy(wtab_hbm.at[widx_v.at[b, j]],
                                 wrows_v.at[b, pl.ds(j * SUB, SUB)], gsem[b])
                pltpu.async_copy(etab_hbm.at[eidx_v.at[b, j]],
                                 erows_v.at[b, pl.ds(j * SUB, SUB)], gsem[b])

        def wait_g(b):
            pltpu.make_async_copy(wtab_hbm.at[pl.ds(0, CHUNK)],
                                  wrows_v.at[b], gsem[b]).wait()
            pltpu.make_async_copy(wtab_hbm.at[pl.ds(0, CHUNK)],
                                  erows_v.at[b], gsem[b]).wait()

        def store(c, b):
            cbase = base + c * CHUNK
            pltpu.async_copy(wrows_v.at[b, :, pl.ds(0, WE_DIM)],
                             out_hbm.at[pl.ds(cbase, CHUNK), pl.ds(0, WE_DIM)],
                             ssem[b])
            pltpu.async_copy(erows_v.at[b, :, pl.ds(0, EE_DIM)],
                             out_hbm.at[pl.ds(cbase, CHUNK), pl.ds(WE_DIM, EE_DIM)],
                             ssem[b])

        def wait_s(b):
            pltpu.make_async_copy(wtab_hbm.at[pl.ds(0, CHUNK // 2)],
                                  wrows_v.at[b, pl.ds(0, CHUNK // 2)],
                                  ssem[b]).wait()
            pltpu.make_async_copy(wtab_hbm.at[pl.ds(0, CHUNK // 2)],
                                  erows_v.at[b, pl.ds(0, CHUNK // 2)],
                                  ssem[b]).wait()

        load_idx(0, 0)
        fire(0, 0)
        load_idx(1, 1)
        fire(1, 1)
        wait_g(0)
        store(0, 0)
        wait_s(0)
        load_idx(2, 0)
        fire(2, 0)
        wait_g(1)
        store(1, 1)

        def body(p, carry):
            c0 = 2 * p
            c1 = c0 + 1
            wait_s(1)
            load_idx(c1, 1)
            fire(c1, 1)
            wait_g(0)
            store(c0, 0)
            wait_s(0)
            load_idx(c0 + 2, 0)
            fire(c0 + 2, 0)
            wait_g(1)
            store(c1, 1)
            return carry

        lax.fori_loop(1, n_pairs - 1, body, 0, unroll=False)

        c0 = n_chunks - 2
        wait_s(1)
        load_idx(c0 + 1, 1)
        fire(c0 + 1, 1)
        wait_g(0)
        store(c0, 0)
        wait_g(1)
        store(c0 + 1, 1)
        wait_s(0)
        wait_s(1)

    return gather_kernel


def kernel(lctx_words, rctx_words, lctx_entities, rctx_entities,
           word_table, entity_table):
    b, l = lctx_words.shape
    n_rows = 2 * b * l
    wv, wd = word_table.shape
    ev, ed = entity_table.shape

    widx = jnp.concatenate(
        [rctx_words.reshape(-1), lctx_words.reshape(-1)]
    ).astype(jnp.int32).reshape(n_rows // SUB, SUB)
    eidx = jnp.concatenate(
        [rctx_entities.reshape(-1), lctx_entities.reshape(-1)]
    ).astype(jnp.int32).reshape(n_rows // SUB, SUB)

    wtab = _make_transpose(wv, wd)(word_table.T)
    etab = _make_transpose(ev, ed)(entity_table.T)
    out = _make_gather(n_rows, wtab.shape[0], etab.shape[0])(
        widx, eidx, wtab, etab)
    return out.reshape(2, b, l, OUT_DIM)


# speedup vs baseline: 2.0498x; 1.0588x over previous
import functools

import jax
import jax.numpy as jnp
from jax import lax
from jax.experimental import pallas as pl
from jax.experimental.pallas import tpu as pltpu
from jax.experimental.pallas import tpu_sc as plsc

WE_DIM = 64
EE_DIM = 64
OUT_DIM = WE_DIM + EE_DIM
TAB_DIM = 128

NC = 2
NS = 16
NW = NC * NS

SUB = 128
CHUNK = 128
NSUB = CHUNK // SUB

L16 = 16


TBLK = 8192


def _make_transpose(v: int, d: int):
    v_pad = ((v + TBLK - 1) // TBLK) * TBLK

    @functools.partial(
        pl.pallas_call,
        grid=(v_pad // TBLK,),
        in_specs=[pl.BlockSpec((d, TBLK), lambda i: (0, i))],
        out_specs=pl.BlockSpec((TBLK, TAB_DIM), lambda i: (i, 0)),
        out_shape=jax.ShapeDtypeStruct((v_pad, TAB_DIM), jnp.float32),
    )
    def transpose_kernel(in_ref, out_ref):
        rows = lax.broadcasted_iota(jnp.int32, (d, d), 0)
        cols = lax.broadcasted_iota(jnp.int32, (d, d), 1)
        eye = jnp.where(rows == cols, 1.0, 0.0).astype(jnp.float32)
        out_ref[:, 0:d] = lax.dot_general(
            in_ref[...], eye, (((0,), (0,)), ((), ())),
            precision=lax.Precision.HIGHEST,
            preferred_element_type=jnp.float32)

    return transpose_kernel


def _make_gather(n_rows: int, wv_pad: int, ev_pad: int):
    rows_per_w = n_rows // NW
    n_chunks = rows_per_w // CHUNK
    n_pairs = n_chunks // 2
    assert n_chunks % 2 == 0 and n_pairs >= 2
    mesh = plsc.VectorSubcoreMesh(core_axis_name="c", subcore_axis_name="s")

    @functools.partial(
        pl.kernel,
        mesh=mesh,
        compiler_params=pltpu.CompilerParams(use_tc_tiling_on_sc=False),
        out_type=jax.ShapeDtypeStruct((n_rows, OUT_DIM), jnp.float32),
        scratch_types=[
            pltpu.VMEM((2, NSUB, SUB), jnp.int32),
            pltpu.VMEM((2, NSUB, SUB), jnp.int32),
            pltpu.VMEM((2, CHUNK, TAB_DIM), jnp.float32),
            pltpu.VMEM((2, CHUNK, TAB_DIM), jnp.float32),
            pltpu.SemaphoreType.DMA,
            pltpu.SemaphoreType.DMA,
            pltpu.SemaphoreType.DMA,
            pltpu.SemaphoreType.DMA,
        ],
    )
    def gather_kernel(widx_hbm, eidx_hbm, wtab_hbm, etab_hbm, out_hbm,
                      widx_v, eidx_v, wrows_v, erows_v,
                      gsem0, gsem1, ssem0, ssem1):
        wid = lax.axis_index("s") * NC + lax.axis_index("c")
        base = wid * rows_per_w
        idx_row0 = wid * (rows_per_w // SUB)
        gsem = (gsem0, gsem1)
        ssem = (ssem0, ssem1)

        def load_idx(c, b):
            crow = idx_row0 + c * NSUB
            pltpu.sync_copy(widx_hbm.at[pl.ds(crow, NSUB)], widx_v.at[b])
            pltpu.sync_copy(eidx_hbm.at[pl.ds(crow, NSUB)], eidx_v.at[b])

        def fire(c, b):
            for j in range(NSUB):
                pltpu.async_copy(wtab_hbm.at[widx_v.at[b, j]],
                                 wrows_v.at[b, pl.ds(j * SUB, SUB)], gsem[b])
                pltpu.async_copy(etab_hbm.at[eidx_v.at[b, j]],
                                 erows_v.at[b, pl.ds(j * SUB, SUB)], gsem[b])

        def wait_g(b):
            pltpu.make_async_copy(wtab_hbm.at[pl.ds(0, CHUNK)],
                                  wrows_v.at[b], gsem[b]).wait()
            pltpu.make_async_copy(wtab_hbm.at[pl.ds(0, CHUNK)],
                                  erows_v.at[b], gsem[b]).wait()

        def store(c, b):
            cbase = base + c * CHUNK
            pltpu.async_copy(wrows_v.at[b, :, pl.ds(0, WE_DIM)],
                             out_hbm.at[pl.ds(cbase, CHUNK), pl.ds(0, WE_DIM)],
                             ssem[b])
            pltpu.async_copy(erows_v.at[b, :, pl.ds(0, EE_DIM)],
                             out_hbm.at[pl.ds(cbase, CHUNK), pl.ds(WE_DIM, EE_DIM)],
                             ssem[b])

        def wait_s(b):
            pltpu.make_async_copy(wtab_hbm.at[pl.ds(0, CHUNK // 2)],
                                  wrows_v.at[b, pl.ds(0, CHUNK // 2)],
                                  ssem[b]).wait()
            pltpu.make_async_copy(wtab_hbm.at[pl.ds(0, CHUNK // 2)],
                                  erows_v.at[b, pl.ds(0, CHUNK // 2)],
                                  ssem[b]).wait()

        load_idx(0, 0)
        fire(0, 0)
        load_idx(1, 1)
        fire(1, 1)
        wait_g(0)
        store(0, 0)
        wait_s(0)
        load_idx(2, 0)
        fire(2, 0)
        wait_g(1)
        store(1, 1)

        def body(p, carry):
            c0 = 2 * p
            c1 = c0 + 1
            wait_s(1)
            load_idx(c1, 1)
            fire(c1, 1)
            wait_g(0)
            store(c0, 0)
            wait_s(0)
            load_idx(c0 + 2, 0)
            fire(c0 + 2, 0)
            wait_g(1)
            store(c1, 1)
            return carry

        lax.fori_loop(1, n_pairs - 1, body, 0, unroll=False)

        c0 = n_chunks - 2
        wait_s(1)
        load_idx(c0 + 1, 1)
        fire(c0 + 1, 1)
        wait_g(0)
        store(c0, 0)
        wait_g(1)
        store(c0 + 1, 1)
        wait_s(0)
        wait_s(1)

    return gather_kernel


def kernel(lctx_words, rctx_words, lctx_entities, rctx_entities,
           word_table, entity_table):
    b, l = lctx_words.shape
    n_rows = 2 * b * l
    wv, wd = word_table.shape
    ev, ed = entity_table.shape

    widx = jnp.concatenate(
        [rctx_words.reshape(-1), lctx_words.reshape(-1)]
    ).astype(jnp.int32).reshape(n_rows // SUB, SUB)
    eidx = jnp.concatenate(
        [rctx_entities.reshape(-1), lctx_entities.reshape(-1)]
    ).astype(jnp.int32).reshape(n_rows // SUB, SUB)

    wtab = _make_transpose(wv, wd)(word_table.T)
    etab = _make_transpose(ev, ed)(entity_table.T)
    out = _make_gather(n_rows, wtab.shape[0], etab.shape[0])(
        widx, eidx, wtab, etab)
    return out.reshape(2, b, l, OUT_DIM)


# speedup vs baseline: 2.2318x; 1.0888x over previous
import functools

import jax
import jax.numpy as jnp
from jax import lax
from jax.experimental import pallas as pl
from jax.experimental.pallas import tpu as pltpu
from jax.experimental.pallas import tpu_sc as plsc

WE_DIM = 64
EE_DIM = 64
OUT_DIM = WE_DIM + EE_DIM
TAB_DIM = 128

NC = 2
NS = 16
NW = NC * NS

SUB = 128
CHUNK = 128
NSUB = CHUNK // SUB

L16 = 16


TBLK = 8192


def _make_transpose(v: int, d: int):
    v_pad = ((v + TBLK - 1) // TBLK) * TBLK

    @functools.partial(
        pl.pallas_call,
        grid=(v_pad // TBLK,),
        in_specs=[pl.BlockSpec((d, TBLK), lambda i: (0, i))],
        out_specs=pl.BlockSpec((TBLK, TAB_DIM), lambda i: (i, 0)),
        out_shape=jax.ShapeDtypeStruct((v_pad, TAB_DIM), jnp.float32),
    )
    def transpose_kernel(in_ref, out_ref):
        rows = lax.broadcasted_iota(jnp.int32, (d, d), 0)
        cols = lax.broadcasted_iota(jnp.int32, (d, d), 1)
        eye = jnp.where(rows == cols, 1.0, 0.0).astype(jnp.float32)
        out_ref[:, 0:d] = lax.dot_general(
            in_ref[...], eye, (((0,), (0,)), ((), ())),
            precision=lax.Precision.HIGHEST,
            preferred_element_type=jnp.float32)

    return transpose_kernel


def _make_gather(n_rows: int, wv_pad: int, ev_pad: int):
    rows_per_w = n_rows // NW
    n_chunks = rows_per_w // CHUNK
    n_pairs = n_chunks // 2
    assert n_chunks % 2 == 0 and n_pairs >= 2
    mesh = plsc.VectorSubcoreMesh(core_axis_name="c", subcore_axis_name="s")

    @functools.partial(
        pl.kernel,
        mesh=mesh,
        compiler_params=pltpu.CompilerParams(use_tc_tiling_on_sc=False),
        out_type=jax.ShapeDtypeStruct((n_rows, OUT_DIM), jnp.float32),
        scratch_types=[
            pltpu.VMEM((rows_per_w // SUB, SUB), jnp.int32),
            pltpu.VMEM((rows_per_w // SUB, SUB), jnp.int32),
            pltpu.VMEM((2, CHUNK, TAB_DIM), jnp.float32),
            pltpu.VMEM((2, CHUNK, TAB_DIM), jnp.float32),
            pltpu.SemaphoreType.DMA,
            pltpu.SemaphoreType.DMA,
            pltpu.SemaphoreType.DMA,
            pltpu.SemaphoreType.DMA,
        ],
    )
    def gather_kernel(widx_hbm, eidx_hbm, wtab_hbm, etab_hbm, out_hbm,
                      widx_v, eidx_v, wrows_v, erows_v,
                      gsem0, gsem1, ssem0, ssem1):
        wid = lax.axis_index("s") * NC + lax.axis_index("c")
        base = wid * rows_per_w
        idx_row0 = wid * (rows_per_w // SUB)
        gsem = (gsem0, gsem1)
        ssem = (ssem0, ssem1)

        def load_idx(c, b):
            del c, b

        def fire(c, b):
            pltpu.async_copy(wtab_hbm.at[widx_v.at[c]],
                             wrows_v.at[b], gsem[b])
            pltpu.async_copy(etab_hbm.at[eidx_v.at[c]],
                             erows_v.at[b], gsem[b])

        def wait_g(b):
            pltpu.make_async_copy(wtab_hbm.at[pl.ds(0, CHUNK)],
                                  wrows_v.at[b], gsem[b]).wait()
            pltpu.make_async_copy(wtab_hbm.at[pl.ds(0, CHUNK)],
                                  erows_v.at[b], gsem[b]).wait()

        def store(c, b):
            cbase = base + c * CHUNK
            pltpu.async_copy(wrows_v.at[b, :, pl.ds(0, WE_DIM)],
                             out_hbm.at[pl.ds(cbase, CHUNK), pl.ds(0, WE_DIM)],
                             ssem[b])
            pltpu.async_copy(erows_v.at[b, :, pl.ds(0, EE_DIM)],
                             out_hbm.at[pl.ds(cbase, CHUNK), pl.ds(WE_DIM, EE_DIM)],
                             ssem[b])

        def wait_s(b):
            pltpu.make_async_copy(wtab_hbm.at[pl.ds(0, CHUNK // 2)],
                                  wrows_v.at[b, pl.ds(0, CHUNK // 2)],
                                  ssem[b]).wait()
            pltpu.make_async_copy(wtab_hbm.at[pl.ds(0, CHUNK // 2)],
                                  erows_v.at[b, pl.ds(0, CHUNK // 2)],
                                  ssem[b]).wait()

        pltpu.sync_copy(widx_hbm.at[pl.ds(idx_row0, rows_per_w // SUB)], widx_v)
        pltpu.sync_copy(eidx_hbm.at[pl.ds(idx_row0, rows_per_w // SUB)], eidx_v)

        load_idx(0, 0)
        fire(0, 0)
        load_idx(1, 1)
        fire(1, 1)
        wait_g(0)
        store(0, 0)
        wait_s(0)
        load_idx(2, 0)
        fire(2, 0)
        wait_g(1)
        store(1, 1)

        def body(p, carry):
            c0 = 2 * p
            c1 = c0 + 1
            wait_s(1)
            load_idx(c1, 1)
            fire(c1, 1)
            wait_g(0)
            store(c0, 0)
            wait_s(0)
            load_idx(c0 + 2, 0)
            fire(c0 + 2, 0)
            wait_g(1)
            store(c1, 1)
            return carry

        lax.fori_loop(1, n_pairs - 1, body, 0, unroll=False)

        c0 = n_chunks - 2
        wait_s(1)
        load_idx(c0 + 1, 1)
        fire(c0 + 1, 1)
        wait_g(0)
        store(c0, 0)
        wait_g(1)
        store(c0 + 1, 1)
        wait_s(0)
        wait_s(1)

    return gather_kernel


def kernel(lctx_words, rctx_words, lctx_entities, rctx_entities,
           word_table, entity_table):
    b, l = lctx_words.shape
    n_rows = 2 * b * l
    wv, wd = word_table.shape
    ev, ed = entity_table.shape

    widx = jnp.concatenate(
        [rctx_words.reshape(-1), lctx_words.reshape(-1)]
    ).astype(jnp.int32).reshape(n_rows // SUB, SUB)
    eidx = jnp.concatenate(
        [rctx_entities.reshape(-1), lctx_entities.reshape(-1)]
    ).astype(jnp.int32).reshape(n_rows // SUB, SUB)

    wtab = _make_transpose(wv, wd)(word_table.T)
    etab = _make_transpose(ev, ed)(entity_table.T)
    out = _make_gather(n_rows, wtab.shape[0], etab.shape[0])(
        widx, eidx, wtab, etab)
    return out.reshape(2, b, l, OUT_DIM)
